# baseline (speedup 1.0000x reference)
import functools

import jax
import jax.numpy as jnp
from jax import lax
from jax.experimental import pallas as pl
from jax.experimental.pallas import tpu as pltpu
from jax.experimental.pallas import tpu_sc as plsc

B = 8
N_CLS = 64
SEQ = 77
N_CTX = 4
D = 512
LANES = 16
NPLANES = SEQ - N_CTX

_info = plsc.get_sparse_core_info()
_NC = _info.num_cores
_NS = _info.num_subcores
_NW = _NC * _NS


def _mlp_body(im_ref, w1_ref, b1_ref, w2_ref, b2_ref, ctx_ref, ub_ref, out_ref):
    h = jnp.maximum(
        jnp.dot(im_ref[...], w1_ref[...], preferred_element_type=jnp.float32)
        + b1_ref[...],
        0.0,
    )
    bias = (
        jnp.dot(h, w2_ref[...], preferred_element_type=jnp.float32) + b2_ref[...]
    ) * ub_ref[0, 0]
    out_ref[...] = ctx_ref[...][None, :, :] + bias[:, None, :]


def _meta_net_ctx(im_features, W1, b1, W2, b2, ctx, use_bias):
    ub = jnp.asarray(use_bias, jnp.float32).reshape(1, 1)
    return pl.pallas_call(
        _mlp_body,
        out_shape=jax.ShapeDtypeStruct((B, N_CTX, D), jnp.float32),
    )(im_features, W1, b1.reshape(1, -1), W2, b2.reshape(1, -1), ctx, ub)


@functools.partial(
    pl.kernel,
    mesh=plsc.VectorSubcoreMesh(core_axis_name="c", subcore_axis_name="s"),
    out_type=jax.ShapeDtypeStruct((NPLANES, N_CLS, D), jnp.float32),
    scratch_types=[
        pltpu.VMEM((1, N_CLS), jnp.int32),
        pltpu.VMEM((N_CLS, D), jnp.float32),
        pltpu.VMEM((N_CLS, D), jnp.float32),
        pltpu.SemaphoreType.DMA,
        pltpu.SemaphoreType.DMA,
        pltpu.SemaphoreType.DMA,
    ],
)
def _sc_gather(table_hbm, tok_hbm, g_hbm, idx_v, buf_a, buf_b,
               gsem, ssem_a, ssem_b):
    wid = lax.axis_index("s") * _NC + lax.axis_index("c")
    bufs = (buf_a, buf_b)
    ssems = (ssem_a, ssem_b)
    outstanding = [jnp.int32(0), jnp.int32(0)]
    for i in range(3):
        p = wid + _NW * i
        cond = p < NPLANES
        slot = i % 2

        def drain(_, slot=slot):
            pltpu.make_async_copy(bufs[slot], g_hbm.at[0], ssems[slot]).wait()
        pl.loop(0, outstanding[slot])(drain)
        outstanding[slot] = jnp.where(cond, 1, 0).astype(jnp.int32)

        @pl.when(cond)
        def _(p=p, slot=slot):
            s = jnp.where(p == 0, 0, p + N_CTX)
            pltpu.sync_copy(tok_hbm.at[s], idx_v)
            pltpu.async_copy(table_hbm.at[idx_v.at[0]], bufs[slot], gsem).wait()
            pltpu.async_copy(bufs[slot], g_hbm.at[p], ssems[slot])

    for slot in (0, 1):
        def drain(_, slot=slot):
            pltpu.make_async_copy(bufs[slot], g_hbm.at[0], ssems[slot]).wait()
        pl.loop(0, outstanding[slot])(drain)


def _tc_body(g_ref, ctx_ref, out_ref):
    s = pl.program_id(0)
    is_ctx = jnp.logical_and(s >= 1, s <= N_CTX)

    @pl.when(is_ctx)
    def _():
        j = jnp.clip(s - 1, 0, N_CTX - 1)
        row = ctx_ref[0, pl.ds(j, 1), :]
        out_ref[...] = jnp.broadcast_to(row, (N_CLS, D))[None, None]

    @pl.when(jnp.logical_not(is_ctx))
    def _():
        out_ref[...] = g_ref[...][None]


def _tc_assemble(G, ctx_shifted):
    def g_map(s, b):
        return (jnp.where(s == 0, 0, jnp.maximum(s - 1 - N_CTX, 0) + 1), 0, 0)

    return pl.pallas_call(
        _tc_body,
        grid=(SEQ, B),
        in_specs=[
            pl.BlockSpec((1, N_CLS, D), g_map),
            pl.BlockSpec((1, N_CTX, D), lambda s, b: (b, 0, 0)),
        ],
        out_specs=pl.BlockSpec((1, 1, N_CLS, D), lambda s, b: (b, s, 0, 0)),
        out_shape=jax.ShapeDtypeStruct((B, SEQ, N_CLS, D), jnp.float32),
    )(G, ctx_shifted)


def kernel(im_features, token_embedding, ctx, W1, b1, W2, b2,
           tokenized_prompts, use_bias=True):
    ctx_shifted = _meta_net_ctx(im_features, W1, b1, W2, b2, ctx, use_bias)
    tok_t = tokenized_prompts.T.reshape(SEQ, 1, N_CLS)
    G = _sc_gather(token_embedding, tok_t)
    out_t = _tc_assemble(G, ctx_shifted)
    special_prompts = jnp.transpose(out_t, (0, 2, 1, 3))
    return (special_prompts, tokenized_prompts)


# speedup vs baseline: 5.0109x; 5.0109x over previous
import functools

import jax
import jax.numpy as jnp
from jax import lax
from jax.experimental import pallas as pl
from jax.experimental.pallas import tpu as pltpu
from jax.experimental.pallas import tpu_sc as plsc

B = 8
N_CLS = 64
SEQ = 77
N_CTX = 4
D = 512
LANES = 16
NPLANES = SEQ - N_CTX

_info = plsc.get_sparse_core_info()
_NC = _info.num_cores
_NS = _info.num_subcores
_NW = _NC * _NS
_MAXP = -(-NPLANES // _NW)


def _mlp_body(im_ref, w1_ref, b1_ref, w2_ref, b2_ref, ctx_ref, ub_ref, out_ref):
    h = jnp.maximum(
        jnp.dot(im_ref[...], w1_ref[...], preferred_element_type=jnp.float32)
        + b1_ref[...],
        0.0,
    )
    bias = (
        jnp.dot(h, w2_ref[...], preferred_element_type=jnp.float32) + b2_ref[...]
    ) * ub_ref[0, 0]
    out = ctx_ref[...][None, :, :] + bias[:, None, :]
    out_ref[...] = out.reshape(B * N_CTX, 1, D)


def _meta_net_ctx(im_features, W1, b1, W2, b2, ctx, use_bias):
    ub = jnp.asarray(use_bias, jnp.float32).reshape(1, 1)
    return pl.pallas_call(
        _mlp_body,
        out_shape=jax.ShapeDtypeStruct((B * N_CTX, 1, D), jnp.float32),
    )(im_features, W1, b1.reshape(1, -1), W2, b2.reshape(1, -1), ctx, ub)


@functools.partial(
    pl.kernel,
    mesh=plsc.VectorSubcoreMesh(core_axis_name="c", subcore_axis_name="s"),
    out_type=jax.ShapeDtypeStruct((B, SEQ, N_CLS, D), jnp.float32),
    scratch_types=[
        pltpu.VMEM((1, N_CLS), jnp.int32),
        pltpu.VMEM((1, D), jnp.float32),
        pltpu.VMEM((N_CLS, D), jnp.float32),
        pltpu.VMEM((N_CLS, D), jnp.float32),
        pltpu.SemaphoreType.DMA,
        pltpu.SemaphoreType.DMA,
        pltpu.SemaphoreType.DMA,
    ],
)
def _sc_assemble(table_hbm, tok_hbm, ctxs_hbm, out_hbm,
                 idx_v, ctxrow_v, buf_a, buf_b,
                 gsem, ssem_a, ssem_b):
    wid = lax.axis_index("s") * _NC + lax.axis_index("c")

    pltpu.sync_copy(ctxs_hbm.at[wid], ctxrow_v)

    def fill_row(r):
        def chunk(k):
            buf_a[r, pl.ds(k * LANES, LANES)] = ctxrow_v[0, pl.ds(k * LANES, LANES)]
        pl.loop(0, D // LANES)(chunk)

    pl.loop(0, N_CLS)(fill_row)
    b_ctx = wid // N_CTX
    j_ctx = wid % N_CTX
    ctx_store = pltpu.async_copy(buf_a, out_hbm.at[b_ctx, 1 + j_ctx], ssem_a)

    bufs = (buf_a, buf_b)
    ssems = (ssem_a, ssem_b)
    tot = NPLANES * B
    lo = (tot * wid) // _NW
    hi = (tot * (wid + 1)) // _NW
    p_base = lo // B
    outstanding = [jnp.int32(1), jnp.int32(0)]
    for i in range(4):
        p = p_base + i
        t_lo = jnp.maximum(lo, p * B)
        t_hi = jnp.minimum(hi, (p + 1) * B)
        n_i = jnp.maximum(t_hi - t_lo, 0)
        slot = i % 2
        buf = bufs[slot]
        sem = ssems[slot]

        def drain(_, buf=buf, sem=sem):
            pltpu.make_async_copy(buf, out_hbm.at[0, 0], sem).wait()
        pl.loop(0, outstanding[slot])(drain)
        outstanding[slot] = n_i

        @pl.when(n_i > 0)
        def _(p=p, t_lo=t_lo, t_hi=t_hi, buf=buf, sem=sem):
            s = jnp.where(p == 0, 0, p + N_CTX)
            pltpu.sync_copy(tok_hbm.at[s], idx_v)
            pltpu.async_copy(table_hbm.at[idx_v.at[0]], buf, gsem).wait()
            def store(b, buf=buf, sem=sem, s=s):
                pltpu.async_copy(buf, out_hbm.at[b, s], sem)
            pl.loop(t_lo - p * B, t_hi - p * B)(store)

    for slot in (0, 1):
        def drain(_, slot=slot):
            pltpu.make_async_copy(bufs[slot], out_hbm.at[0, 0], ssems[slot]).wait()
        pl.loop(0, outstanding[slot])(drain)


def kernel(im_features, token_embedding, ctx, W1, b1, W2, b2,
           tokenized_prompts, use_bias=True):
    ctxs2 = _meta_net_ctx(im_features, W1, b1, W2, b2, ctx, use_bias)
    tok_t = tokenized_prompts.T.reshape(SEQ, 1, N_CLS)
    out_t = _sc_assemble(token_embedding, tok_t, ctxs2)
    special_prompts = jnp.transpose(out_t, (0, 2, 1, 3))
    return (special_prompts, tokenized_prompts)


# speedup vs baseline: 5.1448x; 1.0267x over previous
import functools

import jax
import jax.numpy as jnp
from jax import lax
from jax.experimental import pallas as pl
from jax.experimental.pallas import tpu as pltpu
from jax.experimental.pallas import tpu_sc as plsc

B = 8
N_CLS = 64
SEQ = 77
N_CTX = 4
D = 512
LANES = 16
NPLANES = SEQ - N_CTX

_info = plsc.get_sparse_core_info()
_NC = _info.num_cores
_NS = _info.num_subcores
_NW = _NC * _NS
_MAXP = -(-NPLANES // _NW)


def _mlp_body(im_ref, w1_ref, b1_ref, w2_ref, b2_ref, ctx_ref, ub_ref, out_ref):
    h = jnp.maximum(
        jnp.dot(im_ref[...], w1_ref[...], preferred_element_type=jnp.float32)
        + b1_ref[...],
        0.0,
    )
    bias = (
        jnp.dot(h, w2_ref[...], preferred_element_type=jnp.float32) + b2_ref[...]
    ) * ub_ref[0, 0]
    out = ctx_ref[...][None, :, :] + bias[:, None, :]
    out_ref[...] = out.reshape(B * N_CTX, 1, D)


def _meta_net_ctx(im_features, W1, b1, W2, b2, ctx, use_bias):
    ub = jnp.asarray(use_bias, jnp.float32).reshape(1, 1)
    return pl.pallas_call(
        _mlp_body,
        out_shape=jax.ShapeDtypeStruct((B * N_CTX, 1, D), jnp.float32),
    )(im_features, W1, b1.reshape(1, -1), W2, b2.reshape(1, -1), ctx, ub)


@functools.partial(
    pl.kernel,
    mesh=plsc.VectorSubcoreMesh(core_axis_name="c", subcore_axis_name="s"),
    out_type=jax.ShapeDtypeStruct((B, SEQ, N_CLS, D), jnp.float32),
    scratch_types=[
        pltpu.VMEM((1, N_CLS), jnp.int32),
        pltpu.VMEM((1, D), jnp.float32),
        pltpu.VMEM((N_CLS, D), jnp.float32),
        pltpu.VMEM((N_CLS, D), jnp.float32),
        pltpu.SemaphoreType.DMA,
        pltpu.SemaphoreType.DMA,
        pltpu.SemaphoreType.DMA,
    ],
)
def _sc_assemble(table_hbm, tok_hbm, ctxs_hbm, out_hbm,
                 idx_v, ctxrow_v, buf_a, buf_b,
                 gsem, ssem_a, ssem_b):
    wid = lax.axis_index("s") * _NC + lax.axis_index("c")

    bufs = (buf_a, buf_b)
    ssems = (ssem_a, ssem_b)
    tot = NPLANES * B
    lo = (tot * wid) // _NW
    hi = (tot * (wid + 1)) // _NW
    p_base = lo // B
    s0 = jnp.where(p_base == 0, 0, p_base + N_CTX)
    pltpu.sync_copy(tok_hbm.at[s0], idx_v)
    g0 = pltpu.async_copy(table_hbm.at[idx_v.at[0]], buf_b, gsem)

    pltpu.sync_copy(ctxs_hbm.at[wid], ctxrow_v)

    def fill_row(r):
        def chunk(k):
            buf_a[r, pl.ds(k * LANES, LANES)] = ctxrow_v[0, pl.ds(k * LANES, LANES)]
        pl.loop(0, D // LANES)(chunk)

    pl.loop(0, N_CLS)(fill_row)
    b_ctx = wid // N_CTX
    j_ctx = wid % N_CTX
    pltpu.async_copy(buf_a, out_hbm.at[b_ctx, 1 + j_ctx], ssem_a)

    g0.wait()
    t_hi0 = jnp.minimum(hi, (p_base + 1) * B)

    def store0(b):
        pltpu.async_copy(buf_b, out_hbm.at[b, s0], ssem_b)
    pl.loop(lo - p_base * B, t_hi0 - p_base * B)(store0)

    outstanding = [jnp.int32(1), t_hi0 - lo]
    for i in range(1, 4):
        p = p_base + i
        t_lo = jnp.maximum(lo, p * B)
        t_hi = jnp.minimum(hi, (p + 1) * B)
        n_i = jnp.maximum(t_hi - t_lo, 0)
        slot = (i + 1) % 2
        buf = bufs[slot]
        sem = ssems[slot]

        def drain(_, buf=buf, sem=sem):
            pltpu.make_async_copy(buf, out_hbm.at[0, 0], sem).wait()
        pl.loop(0, outstanding[slot])(drain)
        outstanding[slot] = n_i

        @pl.when(n_i > 0)
        def _(p=p, t_lo=t_lo, t_hi=t_hi, buf=buf, sem=sem):
            s = jnp.where(p == 0, 0, p + N_CTX)
            pltpu.sync_copy(tok_hbm.at[s], idx_v)
            pltpu.async_copy(table_hbm.at[idx_v.at[0]], buf, gsem).wait()
            def store(b, buf=buf, sem=sem, s=s):
                pltpu.async_copy(buf, out_hbm.at[b, s], sem)
            pl.loop(t_lo - p * B, t_hi - p * B)(store)

    for slot in (0, 1):
        def drain(_, slot=slot):
            pltpu.make_async_copy(bufs[slot], out_hbm.at[0, 0], ssems[slot]).wait()
        pl.loop(0, outstanding[slot])(drain)


def kernel(im_features, token_embedding, ctx, W1, b1, W2, b2,
           tokenized_prompts, use_bias=True):
    ctxs2 = _meta_net_ctx(im_features, W1, b1, W2, b2, ctx, use_bias)
    tok_t = tokenized_prompts.T.reshape(SEQ, 1, N_CLS)
    out_t = _sc_assemble(token_embedding, tok_t, ctxs2)
    special_prompts = jnp.transpose(out_t, (0, 2, 1, 3))
    return (special_prompts, tokenized_prompts)
